# Initial kernel scaffold; baseline (speedup 1.0000x reference)
import jax
import jax.numpy as jnp
from jax.experimental import pallas as pl


def kernel(x, edge_index):
    raise NotImplementedError("write your pallas kernel here")


# speedup vs baseline: 8.8096x; 8.8096x over previous
import functools

import jax
import jax.numpy as jnp
from jax import lax
from jax.experimental import pallas as pl
from jax.experimental.pallas import tpu as pltpu
from jax.experimental.pallas import tpu_sc as plsc

_N = 10000
_NP = 10240
_D = 128
_K = 10
_ALPHA = 0.1
_NC = 2
_NT = 16
_NW = _NC * _NT
_CH = 125
_GB = 8
_RC = 128

_RPT = _NP // _NT
_NRC = _RPT // _RC

_mesh = plsc.VectorSubcoreMesh(core_axis_name="c", subcore_axis_name="s")


def _fill_iota(idx_v, base):
    it16 = lax.iota(jnp.int32, 16)
    for c in range(_RC // 16):
        idx_v[pl.ds(c * 16, 16)] = base + c * 16 + it16


@functools.partial(
    pl.kernel,
    out_type=jax.ShapeDtypeStruct((_NC, _NP, 16), jnp.float32),
    mesh=_mesh,
    scratch_types=[
        pltpu.VMEM((_GB, _CH), jnp.int32),
        pltpu.VMEM((_CH, 16), jnp.float32),
        pltpu.VMEM((_RC, 16), jnp.float32),
        pltpu.VMEM((_RC,), jnp.int32),
        pltpu.VMEM_SHARED((_NP, 16), jnp.float32),
        pltpu.SemaphoreType.DMA,
    ],
)
def _deg_kernel(col3d, deg_hbm, coli_v, ones_v, buf_v, idx_v, deg_sp, sem):
    nch = col3d.shape[1]
    ngr = nch // _GB
    cid = lax.axis_index("c")
    sid = lax.axis_index("s")
    wid = sid * _NC + cid
    one16 = jnp.full((16,), 1.0, jnp.float32)
    zero16 = jnp.zeros((16,), jnp.float32)

    def initrow(r, _):
        ones_v[r, pl.ds(0, 16)] = one16
        return 0

    lax.fori_loop(0, _CH, initrow, 0)

    def zrow(r, _):
        buf_v[r, pl.ds(0, 16)] = zero16
        return 0

    lax.fori_loop(0, _RC, zrow, 0)

    def zinit(cc, _):
        _fill_iota(idx_v, sid * _RPT + cc * _RC)
        pltpu.sync_copy(buf_v, deg_sp.at[idx_v])
        return 0

    lax.fori_loop(0, _NRC, zinit, 0)
    plsc.subcore_barrier()

    def group(g, _):
        pltpu.sync_copy(col3d.at[wid, pl.ds(g * _GB, _GB)], coli_v)

        def chunk(j, _):
            pltpu.sync_copy(ones_v, deg_sp.at[coli_v.at[j]], add=True)
            return 0

        lax.fori_loop(0, _GB, chunk, 0)
        return 0

    lax.fori_loop(0, ngr, group, 0)
    plsc.subcore_barrier()

    def wout(cc, _):
        r0 = sid * _RPT + cc * _RC
        _fill_iota(idx_v, r0)
        pltpu.async_copy(deg_sp.at[idx_v], buf_v, sem).wait()
        pltpu.sync_copy(buf_v, deg_hbm.at[cid, pl.ds(r0, _RC)])
        return 0

    lax.fori_loop(0, _NRC, wout, 0)


@functools.partial(
    pl.kernel,
    out_type=jax.ShapeDtypeStruct((_NC, _NP, _D), jnp.float32),
    mesh=_mesh,
    scratch_types=[
        pltpu.VMEM((_GB, _CH), jnp.int32),
        pltpu.VMEM((_GB, _CH), jnp.int32),
        pltpu.VMEM((_CH, _D), jnp.float32),
        pltpu.VMEM((_RC, _D), jnp.float32),
        pltpu.VMEM((_RC,), jnp.int32),
        pltpu.VMEM_SHARED((_NP, _D), jnp.float32),
        pltpu.SemaphoreType.DMA,
    ],
)
def _scatter_kernel(u, halfbias, row3d, col3d, parts_hbm,
                    rowi_v, coli_v, rows_v, buf_v, idx_v, s_sp, sem):
    nch = row3d.shape[1]
    ngr = nch // _GB
    cid = lax.axis_index("c")
    sid = lax.axis_index("s")
    wid = sid * _NC + cid

    def binit(cc, _):
        r0 = sid * _RPT + cc * _RC
        _fill_iota(idx_v, r0)
        pltpu.sync_copy(halfbias.at[pl.ds(r0, _RC)], buf_v)
        pltpu.sync_copy(buf_v, s_sp.at[idx_v])
        return 0

    lax.fori_loop(0, _NRC, binit, 0)
    plsc.subcore_barrier()

    def group(g, _):
        pltpu.sync_copy(row3d.at[wid, pl.ds(g * _GB, _GB)], rowi_v)
        pltpu.sync_copy(col3d.at[wid, pl.ds(g * _GB, _GB)], coli_v)

        def chunk(j, _):
            pltpu.async_copy(u.at[rowi_v.at[j]], rows_v, sem).wait()
            pltpu.sync_copy(rows_v, s_sp.at[coli_v.at[j]], add=True)
            return 0

        lax.fori_loop(0, _GB, chunk, 0)
        return 0

    lax.fori_loop(0, ngr, group, 0)
    plsc.subcore_barrier()

    def wout(cc, _):
        r0 = sid * _RPT + cc * _RC
        _fill_iota(idx_v, r0)
        pltpu.async_copy(s_sp.at[idx_v], buf_v, sem).wait()
        pltpu.sync_copy(buf_v, parts_hbm.at[cid, pl.ds(r0, _RC)])
        return 0

    lax.fori_loop(0, _NRC, wout, 0)


def _combine_body(parts_ref, a2_ref, out_ref):
    out_ref[...] = a2_ref[...] * (parts_ref[0] + parts_ref[1])


_combine = pl.pallas_call(
    _combine_body,
    out_shape=jax.ShapeDtypeStruct((_NP, _D), jnp.float32),
    grid=(_NP // 256,),
    in_specs=[
        pl.BlockSpec((_NC, 256, _D), lambda i: (0, i, 0)),
        pl.BlockSpec((256, 1), lambda i: (i, 0)),
    ],
    out_specs=pl.BlockSpec((256, _D), lambda i: (i, 0)),
)


def kernel(x, edge_index):
    n, d = x.shape
    e = edge_index.shape[1]
    assert n == _N and d == _D
    assert e % (_NW * _CH * _GB) == 0
    row = edge_index[0]
    col = edge_index[1]
    epw = e // _NW
    row3d = row.reshape(_NW, epw // _CH, _CH)
    col3d = col.reshape(_NW, epw // _CH, _CH)

    degp = _deg_kernel(col3d).sum(axis=0)[:, 0]
    posp = degp > 0.0
    disp = jnp.where(posp, lax.rsqrt(jnp.maximum(degp, 1e-12)), 0.0)
    u0 = jnp.pad(x, ((0, _NP - n), (0, 0))) * disp[:, None]
    a2 = ((1.0 - _ALPHA) * disp * disp)[:, None]
    halfbias = (0.5 * _ALPHA / (1.0 - _ALPHA)) * u0 * degp[:, None]

    u = u0
    for _ in range(_K):
        parts = _scatter_kernel(u, halfbias, row3d, col3d)
        u = _combine(parts, a2)

    deg = degp[:n]
    h = jnp.where(posp[:n, None],
                  (u[:n] - _ALPHA * u0[:n]) * jnp.sqrt(deg)[:, None],
                  0.0) + _ALPHA * x
    return h
